# Initial kernel scaffold; baseline (speedup 1.0000x reference)
import jax
import jax.numpy as jnp
from jax.experimental import pallas as pl


def kernel(x, edge_index, W_l1, b_l1, W_r1, b_r1, att1, bias1, ln_g1, ln_b1, W_l2, b_l2, W_r2, b_r2, att2, bias2, ln_g2, ln_b2, W_l3, b_l3, W_r3, b_r3, att3, bias3, ln_g3, ln_b3):
    raise NotImplementedError("write your pallas kernel here")


# speedup vs baseline: 31.2981x; 31.2981x over previous
import functools

import jax
import jax.numpy as jnp
from jax import lax
from jax.experimental import pallas as pl
from jax.experimental.pallas import tpu as pltpu
from jax.experimental.pallas import tpu_sc as plsc

N_NODES = 10000
N_PAD = 10240
D = 128
ACCW = 144
NC, NS, L = 2, 16, 16
NW = NC * NS
K = 48
ROWS_PER_SUB = N_PAD // NS
ZCH = 40
WCH = 128


def _make_edge_kernel(heads, e_real, epw):
    cpw = epw // K
    vph = (D // L) // heads
    mesh = plsc.VectorSubcoreMesh(core_axis_name="c", subcore_axis_name="s")

    @functools.partial(
        pl.kernel,
        mesh=mesh,
        out_type=jax.ShapeDtypeStruct((NC, N_PAD, ACCW), jnp.float32),
        compiler_params=pltpu.CompilerParams(
            use_tc_tiling_on_sc=False, needs_layout_passes=False),
        scratch_types=[
            pltpu.VMEM((2, K), jnp.int32),
            pltpu.VMEM((2, K), jnp.int32),
            pltpu.VMEM((K, D), jnp.float32),
            pltpu.VMEM((K, D), jnp.float32),
            pltpu.VMEM((K, D), jnp.float32),
            pltpu.VMEM((K, D), jnp.float32),
            pltpu.VMEM((K, ACCW), jnp.float32),
            pltpu.VMEM((D,), jnp.float32),
            pltpu.VMEM_SHARED((N_PAD, ACCW), jnp.float32),
            pltpu.SemaphoreType.DMA,
            pltpu.SemaphoreType.DMA,
            pltpu.SemaphoreType.DMA,
            pltpu.SemaphoreType.DMA,
        ],
    )
    def edge_kernel(edges_hbm, tabl_hbm, tabr_hbm, att_hbm,
                    out_hbm, idx0_v, idx1_v, xl0_v, xl1_v, xr0_v, xr1_v,
                    orow_v, att_v, acc_sh, sl0, sl1, sr0, sr1):
        cid = lax.axis_index("c")
        sid = lax.axis_index("s")
        wid = cid * NS + sid
        idx_v = (idx0_v, idx1_v)
        xl_v = (xl0_v, xl1_v)
        xr_v = (xr0_v, xr1_v)
        sl = (sl0, sl1)
        sr = (sr0, sr1)

        zv = jnp.zeros((L,), jnp.float32)

        def zrow(i, carry):
            for j in range(ACCW // L):
                orow_v[i, pl.ds(j * L, L)] = zv
            return carry

        lax.fori_loop(0, ZCH, zrow, 0)
        row0 = sid * ROWS_PER_SUB
        for t in range(ROWS_PER_SUB // ZCH):
            pltpu.sync_copy(orow_v.at[pl.ds(0, ZCH)],
                            acc_sh.at[pl.ds(row0 + t * ZCH, ZCH)])
        plsc.subcore_barrier()

        pltpu.sync_copy(att_hbm, att_v)
        attv = [att_v[pl.ds(j * L, L)] for j in range(D // L)]
        lane = jnp.arange(L, dtype=jnp.int32)
        ebase = wid * epw

        def fetch(g, b):
            base = ebase + g * K
            pltpu.sync_copy(edges_hbm.at[:, pl.ds(base, K)], idx_v[b])
            pltpu.async_copy(tabl_hbm.at[idx_v[b].at[0]], xl_v[b], sl[b])
            pltpu.async_copy(tabr_hbm.at[idx_v[b].at[1]], xr_v[b], sr[b])

        def compute(g, b):
            pltpu.make_async_copy(
                tabl_hbm.at[idx_v[b].at[0]], xl_v[b], sl[b]).wait()
            pltpu.make_async_copy(
                tabr_hbm.at[idx_v[b].at[1]], xr_v[b], sr[b]).wait()
            base = ebase + g * K

            def edge_body(e, ecarry):
                xl = [xl_v[b][e, pl.ds(j * L, L)] for j in range(D // L)]
                t = []
                for j in range(D // L):
                    s = xl[j] + xr_v[b][e, pl.ds(j * L, L)]
                    t.append(jnp.maximum(s, s * 0.2) * attv[j])
                valid = (base + e) < e_real
                pv = jnp.zeros((L,), jnp.float32)
                pbs = []
                for h in range(heads):
                    acc = t[h * vph]
                    for j in range(1, vph):
                        acc = acc + t[h * vph + j]
                    alpha = jnp.sum(acc)
                    alpha = jnp.where(valid, alpha, -1e30)
                    pb = jnp.exp(jnp.broadcast_to(alpha, (L,)))
                    pbs.append(pb)
                    pv = jnp.where(lane == h, pb, pv)
                for j in range(D // L):
                    orow_v[e, pl.ds(j * L, L)] = xl[j] * pbs[j // vph]
                orow_v[e, pl.ds(D, L)] = pv
                return ecarry

            lax.fori_loop(0, K, edge_body, 0)
            pltpu.sync_copy(orow_v, acc_sh.at[idx_v[b].at[1]], add=True)

        fetch(0, 0)

        def pair_body(i, carry):
            g0 = 2 * i
            fetch(g0 + 1, 1)
            compute(g0, 0)
            fetch(jnp.minimum(g0 + 2, cpw - 1), 0)
            compute(g0 + 1, 1)
            return carry

        lax.fori_loop(0, cpw // 2, pair_body, 0)
        pltpu.make_async_copy(
            tabl_hbm.at[idx_v[0].at[0]], xl_v[0], sl[0]).wait()
        pltpu.make_async_copy(
            tabr_hbm.at[idx_v[0].at[1]], xr_v[0], sr[0]).wait()
        plsc.subcore_barrier()

        for t in range(ROWS_PER_SUB // WCH):
            r = row0 + t * WCH
            pltpu.sync_copy(acc_sh.at[pl.ds(r, WCH)],
                            out_hbm.at[cid, pl.ds(r, WCH)])

    return edge_kernel


_MM_BLK = 400


def _mm_body(x_ref, w_ref, b_ref, ol_ref, or_ref):
    y = jnp.dot(x_ref[...], w_ref[...], preferred_element_type=jnp.float32)
    y = y + b_ref[...]
    ol_ref[...] = y[:, :D]
    or_ref[...] = y[:, D:]


def _mm_call(x, w, b):
    grid = N_NODES // _MM_BLK
    return pl.pallas_call(
        _mm_body,
        grid=(grid,),
        in_specs=[
            pl.BlockSpec((_MM_BLK, D), lambda i: (i, 0)),
            pl.BlockSpec((D, 2 * D), lambda i: (0, 0)),
            pl.BlockSpec((1, 2 * D), lambda i: (0, 0)),
        ],
        out_specs=[
            pl.BlockSpec((_MM_BLK, D), lambda i: (i, 0)),
            pl.BlockSpec((_MM_BLK, D), lambda i: (i, 0)),
        ],
        out_shape=[
            jax.ShapeDtypeStruct((N_NODES, D), jnp.float32),
            jax.ShapeDtypeStruct((N_NODES, D), jnp.float32),
        ],
    )(x, w, b)


def _comb_body(acc_ref, bias_ref, g_ref, b_ref, o_ref, *, heads):
    s = acc_ref[0] + acc_ref[1]
    cph = D // heads
    parts = []
    for h in range(heads):
        den = s[:, D + h:D + h + 1] + 1e-16
        parts.append(s[:, h * cph:(h + 1) * cph] / den)
    y = jnp.concatenate(parts, axis=1) + bias_ref[...]
    mu = jnp.mean(y, axis=-1, keepdims=True)
    d = y - mu
    var = jnp.mean(d * d, axis=-1, keepdims=True)
    z = d / jnp.sqrt(var + 1e-5) * g_ref[...] + b_ref[...]
    o_ref[...] = jnp.maximum(z, 0.0)


def _comb_call(acc, bias, g, b, heads):
    grid = N_NODES // _MM_BLK
    return pl.pallas_call(
        functools.partial(_comb_body, heads=heads),
        grid=(grid,),
        in_specs=[
            pl.BlockSpec((2, _MM_BLK, ACCW), lambda i: (0, i, 0)),
            pl.BlockSpec((1, D), lambda i: (0, 0)),
            pl.BlockSpec((1, D), lambda i: (0, 0)),
            pl.BlockSpec((1, D), lambda i: (0, 0)),
        ],
        out_specs=pl.BlockSpec((_MM_BLK, D), lambda i: (i, 0)),
        out_shape=jax.ShapeDtypeStruct((N_NODES, D), jnp.float32),
    )(acc, bias, g, b)


def kernel(x, edge_index, W_l1, b_l1, W_r1, b_r1, att1, bias1, ln_g1, ln_b1,
           W_l2, b_l2, W_r2, b_r2, att2, bias2, ln_g2, ln_b2,
           W_l3, b_l3, W_r3, b_r3, att3, bias3, ln_g3, ln_b3):
    e_in = edge_index.shape[1]
    e_real = e_in + N_NODES
    epw = -(-e_real // (NW * 2 * K)) * 2 * K
    epad = NW * epw

    loops = jnp.arange(N_NODES, dtype=jnp.int32)
    src = jnp.concatenate([edge_index[0].astype(jnp.int32), loops])
    dst = jnp.concatenate([edge_index[1].astype(jnp.int32), loops])
    edges = jnp.stack([jnp.pad(src, (0, epad - e_real)),
                       jnp.pad(dst, (0, epad - e_real))])

    ek4 = _make_edge_kernel(4, e_real, epw)
    ek1 = _make_edge_kernel(1, e_real, epw)

    h = x
    layers = [
        (4, ek4, W_l1, b_l1, W_r1, b_r1, att1, bias1, ln_g1, ln_b1),
        (4, ek4, W_l2, b_l2, W_r2, b_r2, att2, bias2, ln_g2, ln_b2),
        (1, ek1, W_l3, b_l3, W_r3, b_r3, att3, bias3, ln_g3, ln_b3),
    ]
    for heads, ek, Wl, bl, Wr, br, att, bias, g, b in layers:
        w = jnp.concatenate([Wl, Wr], axis=1)
        bb = jnp.concatenate([bl, br]).reshape(1, 2 * D)
        xl, xr = _mm_call(h, w, bb)
        acc = ek(edges, xl, xr, att.reshape(-1))
        h = _comb_call(acc, bias.reshape(1, D), g.reshape(1, D),
                       b.reshape(1, D), heads)
    return h


# speedup vs baseline: 46.4852x; 1.4852x over previous
import functools

import jax
import jax.numpy as jnp
from jax import lax
from jax.experimental import pallas as pl
from jax.experimental.pallas import tpu as pltpu
from jax.experimental.pallas import tpu_sc as plsc

N_NODES = 10000
N_PAD = 10240
D = 128
ACCW = 144
NC, NS, L = 2, 16, 16
NW = NC * NS
K = 48
ROWS_PER_SUB = N_PAD // NS
ZCH = 40
WCH = 128


def _make_edge_kernel(heads, e_real, epw):
    cpw = epw // K
    vph = (D // L) // heads
    mesh = plsc.VectorSubcoreMesh(core_axis_name="c", subcore_axis_name="s")

    @functools.partial(
        pl.kernel,
        mesh=mesh,
        out_type=jax.ShapeDtypeStruct((NC, N_PAD, ACCW), jnp.float32),
        compiler_params=pltpu.CompilerParams(
            use_tc_tiling_on_sc=False, needs_layout_passes=False),
        scratch_types=[
            pltpu.VMEM((2, K), jnp.int32),
            pltpu.VMEM((2, K), jnp.int32),
            pltpu.VMEM((K, D), jnp.float32),
            pltpu.VMEM((K, D), jnp.float32),
            pltpu.VMEM((K, D), jnp.float32),
            pltpu.VMEM((K, D), jnp.float32),
            pltpu.VMEM((K, ACCW), jnp.float32),
            pltpu.VMEM((D,), jnp.float32),
            pltpu.VMEM_SHARED((N_PAD, ACCW), jnp.float32),
            pltpu.SemaphoreType.DMA,
            pltpu.SemaphoreType.DMA,
            pltpu.SemaphoreType.DMA,
            pltpu.SemaphoreType.DMA,
        ],
    )
    def edge_kernel(edges_hbm, tabl_hbm, tabr_hbm, att_hbm,
                    out_hbm, idx0_v, idx1_v, xl0_v, xl1_v, xr0_v, xr1_v,
                    orow_v, att_v, acc_sh, sl0, sl1, sr0, sr1):
        cid = lax.axis_index("c")
        sid = lax.axis_index("s")
        wid = cid * NS + sid
        idx_v = (idx0_v, idx1_v)
        xl_v = (xl0_v, xl1_v)
        xr_v = (xr0_v, xr1_v)
        sl = (sl0, sl1)
        sr = (sr0, sr1)

        zv = jnp.zeros((L,), jnp.float32)

        def zrow(i, carry):
            for j in range(ACCW // L):
                orow_v[i, pl.ds(j * L, L)] = zv
            return carry

        lax.fori_loop(0, ZCH, zrow, 0)
        row0 = sid * ROWS_PER_SUB
        for t in range(ROWS_PER_SUB // ZCH):
            pltpu.sync_copy(orow_v.at[pl.ds(0, ZCH)],
                            acc_sh.at[pl.ds(row0 + t * ZCH, ZCH)])
        plsc.subcore_barrier()

        pltpu.sync_copy(att_hbm, att_v)
        attv = [att_v[pl.ds(j * L, L)] for j in range(D // L)]
        lane = jnp.arange(L, dtype=jnp.int32)
        ebase = wid * epw

        def fetch(g, b):
            base = ebase + g * K
            pltpu.sync_copy(edges_hbm.at[:, pl.ds(base, K)], idx_v[b])
            pltpu.async_copy(tabl_hbm.at[idx_v[b].at[0]], xl_v[b], sl[b])
            pltpu.async_copy(tabr_hbm.at[idx_v[b].at[1]], xr_v[b], sr[b])

        def compute(g, b):
            pltpu.make_async_copy(
                tabl_hbm.at[idx_v[b].at[0]], xl_v[b], sl[b]).wait()
            pltpu.make_async_copy(
                tabr_hbm.at[idx_v[b].at[1]], xr_v[b], sr[b]).wait()
            base = ebase + g * K

            @plsc.parallel_loop(0, K, unroll=4)
            def edge_body(e):
                xl = [xl_v[b][e, pl.ds(j * L, L)] for j in range(D // L)]
                t = []
                for j in range(D // L):
                    s = xl[j] + xr_v[b][e, pl.ds(j * L, L)]
                    t.append(jnp.maximum(s, s * 0.2) * attv[j])
                valid = (base + e) < e_real
                pv = jnp.zeros((L,), jnp.float32)
                pbs = []
                for h in range(heads):
                    acc = t[h * vph]
                    for j in range(1, vph):
                        acc = acc + t[h * vph + j]
                    alpha = jnp.sum(acc)
                    alpha = jnp.where(valid, alpha, -1e30)
                    pb = jnp.exp(jnp.broadcast_to(alpha, (L,)))
                    pbs.append(pb)
                    pv = jnp.where(lane == h, pb, pv)
                for j in range(D // L):
                    orow_v[e, pl.ds(j * L, L)] = xl[j] * pbs[j // vph]
                orow_v[e, pl.ds(D, L)] = pv

            pltpu.sync_copy(orow_v, acc_sh.at[idx_v[b].at[1]], add=True)

        fetch(0, 0)

        def pair_body(i, carry):
            g0 = 2 * i
            fetch(g0 + 1, 1)
            compute(g0, 0)
            fetch(jnp.minimum(g0 + 2, cpw - 1), 0)
            compute(g0 + 1, 1)
            return carry

        lax.fori_loop(0, cpw // 2, pair_body, 0)
        pltpu.make_async_copy(
            tabl_hbm.at[idx_v[0].at[0]], xl_v[0], sl[0]).wait()
        pltpu.make_async_copy(
            tabr_hbm.at[idx_v[0].at[1]], xr_v[0], sr[0]).wait()
        plsc.subcore_barrier()

        for t in range(ROWS_PER_SUB // WCH):
            r = row0 + t * WCH
            pltpu.sync_copy(acc_sh.at[pl.ds(r, WCH)],
                            out_hbm.at[cid, pl.ds(r, WCH)])

    return edge_kernel


_MM_BLK = 400


def _mm_body(x_ref, w_ref, b_ref, ol_ref, or_ref):
    y = jnp.dot(x_ref[...], w_ref[...], preferred_element_type=jnp.float32)
    y = y + b_ref[...]
    ol_ref[...] = y[:, :D]
    or_ref[...] = y[:, D:]


def _mm_call(x, w, b):
    grid = N_NODES // _MM_BLK
    return pl.pallas_call(
        _mm_body,
        grid=(grid,),
        in_specs=[
            pl.BlockSpec((_MM_BLK, D), lambda i: (i, 0)),
            pl.BlockSpec((D, 2 * D), lambda i: (0, 0)),
            pl.BlockSpec((1, 2 * D), lambda i: (0, 0)),
        ],
        out_specs=[
            pl.BlockSpec((_MM_BLK, D), lambda i: (i, 0)),
            pl.BlockSpec((_MM_BLK, D), lambda i: (i, 0)),
        ],
        out_shape=[
            jax.ShapeDtypeStruct((N_NODES, D), jnp.float32),
            jax.ShapeDtypeStruct((N_NODES, D), jnp.float32),
        ],
    )(x, w, b)


def _comb_body(acc_ref, bias_ref, g_ref, b_ref, o_ref, *, heads):
    s = acc_ref[0] + acc_ref[1]
    cph = D // heads
    parts = []
    for h in range(heads):
        den = s[:, D + h:D + h + 1] + 1e-16
        parts.append(s[:, h * cph:(h + 1) * cph] / den)
    y = jnp.concatenate(parts, axis=1) + bias_ref[...]
    mu = jnp.mean(y, axis=-1, keepdims=True)
    d = y - mu
    var = jnp.mean(d * d, axis=-1, keepdims=True)
    z = d / jnp.sqrt(var + 1e-5) * g_ref[...] + b_ref[...]
    o_ref[...] = jnp.maximum(z, 0.0)


def _comb_call(acc, bias, g, b, heads):
    grid = N_NODES // _MM_BLK
    return pl.pallas_call(
        functools.partial(_comb_body, heads=heads),
        grid=(grid,),
        in_specs=[
            pl.BlockSpec((2, _MM_BLK, ACCW), lambda i: (0, i, 0)),
            pl.BlockSpec((1, D), lambda i: (0, 0)),
            pl.BlockSpec((1, D), lambda i: (0, 0)),
            pl.BlockSpec((1, D), lambda i: (0, 0)),
        ],
        out_specs=pl.BlockSpec((_MM_BLK, D), lambda i: (i, 0)),
        out_shape=jax.ShapeDtypeStruct((N_NODES, D), jnp.float32),
    )(acc, bias, g, b)


def kernel(x, edge_index, W_l1, b_l1, W_r1, b_r1, att1, bias1, ln_g1, ln_b1,
           W_l2, b_l2, W_r2, b_r2, att2, bias2, ln_g2, ln_b2,
           W_l3, b_l3, W_r3, b_r3, att3, bias3, ln_g3, ln_b3):
    e_in = edge_index.shape[1]
    e_real = e_in + N_NODES
    epw = -(-e_real // (NW * 2 * K)) * 2 * K
    epad = NW * epw

    loops = jnp.arange(N_NODES, dtype=jnp.int32)
    src = jnp.concatenate([edge_index[0].astype(jnp.int32), loops])
    dst = jnp.concatenate([edge_index[1].astype(jnp.int32), loops])
    edges = jnp.stack([jnp.pad(src, (0, epad - e_real)),
                       jnp.pad(dst, (0, epad - e_real))])

    ek4 = _make_edge_kernel(4, e_real, epw)
    ek1 = _make_edge_kernel(1, e_real, epw)

    h = x
    layers = [
        (4, ek4, W_l1, b_l1, W_r1, b_r1, att1, bias1, ln_g1, ln_b1),
        (4, ek4, W_l2, b_l2, W_r2, b_r2, att2, bias2, ln_g2, ln_b2),
        (1, ek1, W_l3, b_l3, W_r3, b_r3, att3, bias3, ln_g3, ln_b3),
    ]
    for heads, ek, Wl, bl, Wr, br, att, bias, g, b in layers:
        w = jnp.concatenate([Wl, Wr], axis=1)
        bb = jnp.concatenate([bl, br]).reshape(1, 2 * D)
        xl, xr = _mm_call(h, w, bb)
        acc = ek(edges, xl, xr, att.reshape(-1))
        h = _comb_call(acc, bias.reshape(1, D), g.reshape(1, D),
                       b.reshape(1, D), heads)
    return h


# speedup vs baseline: 48.5147x; 1.0437x over previous
import functools

import jax
import jax.numpy as jnp
from jax import lax
from jax.experimental import pallas as pl
from jax.experimental.pallas import tpu as pltpu
from jax.experimental.pallas import tpu_sc as plsc

N_NODES = 10000
N_PAD = 10240
D = 128
ACCW = 144
NC, NS, L = 2, 16, 16
NW = NC * NS
K = 48
ROWS_PER_SUB = N_PAD // NS
ZCH = 40
WCH = 128


def _make_edge_kernel(heads, e_real, epw):
    cpw = epw // K
    vph = (D // L) // heads
    mesh = plsc.VectorSubcoreMesh(core_axis_name="c", subcore_axis_name="s")

    @functools.partial(
        pl.kernel,
        mesh=mesh,
        out_type=jax.ShapeDtypeStruct((NC, N_PAD, ACCW), jnp.float32),
        compiler_params=pltpu.CompilerParams(
            use_tc_tiling_on_sc=False, needs_layout_passes=False),
        scratch_types=[
            pltpu.VMEM((2, K), jnp.int32),
            pltpu.VMEM((2, K), jnp.int32),
            pltpu.VMEM((K, D), jnp.float32),
            pltpu.VMEM((K, D), jnp.float32),
            pltpu.VMEM((K, D), jnp.float32),
            pltpu.VMEM((K, D), jnp.float32),
            pltpu.VMEM((K, ACCW), jnp.float32),
            pltpu.VMEM((D,), jnp.float32),
            pltpu.VMEM_SHARED((N_PAD, ACCW), jnp.float32),
            pltpu.SemaphoreType.DMA,
            pltpu.SemaphoreType.DMA,
            pltpu.SemaphoreType.DMA,
            pltpu.SemaphoreType.DMA,
        ],
    )
    def edge_kernel(edges_hbm, tabl_hbm, tabr_hbm, att_hbm,
                    out_hbm, idx0_v, idx1_v, xl0_v, xl1_v, xr0_v, xr1_v,
                    orow_v, att_v, acc_sh, sl0, sl1, sr0, sr1):
        cid = lax.axis_index("c")
        sid = lax.axis_index("s")
        wid = cid * NS + sid
        idx_v = (idx0_v, idx1_v)
        xl_v = (xl0_v, xl1_v)
        xr_v = (xr0_v, xr1_v)
        sl = (sl0, sl1)
        sr = (sr0, sr1)

        zv = jnp.zeros((L,), jnp.float32)

        def zrow(i, carry):
            for j in range(ACCW // L):
                orow_v[i, pl.ds(j * L, L)] = zv
            return carry

        lax.fori_loop(0, ZCH, zrow, 0)
        row0 = sid * ROWS_PER_SUB
        for t in range(ROWS_PER_SUB // ZCH):
            pltpu.sync_copy(orow_v.at[pl.ds(0, ZCH)],
                            acc_sh.at[pl.ds(row0 + t * ZCH, ZCH)])
        plsc.subcore_barrier()

        pltpu.sync_copy(att_hbm, att_v)
        attv = [att_v[pl.ds(j * L, L)] for j in range(D // L)]
        lane = jnp.arange(L, dtype=jnp.int32)
        ebase = wid * epw

        def fetch(g, b):
            base = ebase + g * K
            pltpu.sync_copy(edges_hbm.at[:, pl.ds(base, K)], idx_v[b])
            pltpu.async_copy(tabl_hbm.at[idx_v[b].at[0]], xl_v[b], sl[b])
            pltpu.async_copy(tabr_hbm.at[idx_v[b].at[1]], xr_v[b], sr[b])

        def compute(g, b):
            pltpu.make_async_copy(
                tabl_hbm.at[idx_v[b].at[0]], xl_v[b], sl[b]).wait()
            pltpu.make_async_copy(
                tabr_hbm.at[idx_v[b].at[1]], xr_v[b], sr[b]).wait()
            base = ebase + g * K

            @plsc.parallel_loop(0, K, unroll=8)
            def edge_body(e):
                xl = [xl_v[b][e, pl.ds(j * L, L)] for j in range(D // L)]
                t = []
                for j in range(D // L):
                    s = xl[j] + xr_v[b][e, pl.ds(j * L, L)]
                    t.append(jnp.maximum(s, s * 0.2) * attv[j])
                valid = (base + e) < e_real
                pv = jnp.zeros((L,), jnp.float32)
                pbs = []
                for h in range(heads):
                    acc = t[h * vph]
                    for j in range(1, vph):
                        acc = acc + t[h * vph + j]
                    alpha = jnp.sum(acc)
                    alpha = jnp.where(valid, alpha, -1e30)
                    pb = jnp.exp(jnp.broadcast_to(alpha, (L,)))
                    pbs.append(pb)
                    pv = jnp.where(lane == h, pb, pv)
                for j in range(D // L):
                    orow_v[e, pl.ds(j * L, L)] = xl[j] * pbs[j // vph]
                orow_v[e, pl.ds(D, L)] = pv

            pltpu.sync_copy(orow_v, acc_sh.at[idx_v[b].at[1]], add=True)

        fetch(0, 0)

        def pair_body(i, carry):
            g0 = 2 * i
            fetch(g0 + 1, 1)
            compute(g0, 0)
            fetch(jnp.minimum(g0 + 2, cpw - 1), 0)
            compute(g0 + 1, 1)
            return carry

        lax.fori_loop(0, cpw // 2, pair_body, 0)
        pltpu.make_async_copy(
            tabl_hbm.at[idx_v[0].at[0]], xl_v[0], sl[0]).wait()
        pltpu.make_async_copy(
            tabr_hbm.at[idx_v[0].at[1]], xr_v[0], sr[0]).wait()
        plsc.subcore_barrier()

        for t in range(ROWS_PER_SUB // WCH):
            r = row0 + t * WCH
            pltpu.sync_copy(acc_sh.at[pl.ds(r, WCH)],
                            out_hbm.at[cid, pl.ds(r, WCH)])

    return edge_kernel


_MM_BLK = 400


def _mm_body(x_ref, w_ref, b_ref, ol_ref, or_ref):
    y = jnp.dot(x_ref[...], w_ref[...], preferred_element_type=jnp.float32)
    y = y + b_ref[...]
    ol_ref[...] = y[:, :D]
    or_ref[...] = y[:, D:]


def _mm_call(x, w, b):
    grid = N_NODES // _MM_BLK
    return pl.pallas_call(
        _mm_body,
        grid=(grid,),
        in_specs=[
            pl.BlockSpec((_MM_BLK, D), lambda i: (i, 0)),
            pl.BlockSpec((D, 2 * D), lambda i: (0, 0)),
            pl.BlockSpec((1, 2 * D), lambda i: (0, 0)),
        ],
        out_specs=[
            pl.BlockSpec((_MM_BLK, D), lambda i: (i, 0)),
            pl.BlockSpec((_MM_BLK, D), lambda i: (i, 0)),
        ],
        out_shape=[
            jax.ShapeDtypeStruct((N_NODES, D), jnp.float32),
            jax.ShapeDtypeStruct((N_NODES, D), jnp.float32),
        ],
    )(x, w, b)


def _comb_body(acc_ref, bias_ref, g_ref, b_ref, o_ref, *, heads):
    s = acc_ref[0] + acc_ref[1]
    cph = D // heads
    parts = []
    for h in range(heads):
        den = s[:, D + h:D + h + 1] + 1e-16
        parts.append(s[:, h * cph:(h + 1) * cph] / den)
    y = jnp.concatenate(parts, axis=1) + bias_ref[...]
    mu = jnp.mean(y, axis=-1, keepdims=True)
    d = y - mu
    var = jnp.mean(d * d, axis=-1, keepdims=True)
    z = d / jnp.sqrt(var + 1e-5) * g_ref[...] + b_ref[...]
    o_ref[...] = jnp.maximum(z, 0.0)


def _comb_call(acc, bias, g, b, heads):
    grid = N_NODES // _MM_BLK
    return pl.pallas_call(
        functools.partial(_comb_body, heads=heads),
        grid=(grid,),
        in_specs=[
            pl.BlockSpec((2, _MM_BLK, ACCW), lambda i: (0, i, 0)),
            pl.BlockSpec((1, D), lambda i: (0, 0)),
            pl.BlockSpec((1, D), lambda i: (0, 0)),
            pl.BlockSpec((1, D), lambda i: (0, 0)),
        ],
        out_specs=pl.BlockSpec((_MM_BLK, D), lambda i: (i, 0)),
        out_shape=jax.ShapeDtypeStruct((N_NODES, D), jnp.float32),
    )(acc, bias, g, b)


def kernel(x, edge_index, W_l1, b_l1, W_r1, b_r1, att1, bias1, ln_g1, ln_b1,
           W_l2, b_l2, W_r2, b_r2, att2, bias2, ln_g2, ln_b2,
           W_l3, b_l3, W_r3, b_r3, att3, bias3, ln_g3, ln_b3):
    e_in = edge_index.shape[1]
    e_real = e_in + N_NODES
    epw = -(-e_real // (NW * 2 * K)) * 2 * K
    epad = NW * epw

    loops = jnp.arange(N_NODES, dtype=jnp.int32)
    src = jnp.concatenate([edge_index[0].astype(jnp.int32), loops])
    dst = jnp.concatenate([edge_index[1].astype(jnp.int32), loops])
    edges = jnp.stack([jnp.pad(src, (0, epad - e_real)),
                       jnp.pad(dst, (0, epad - e_real))])

    ek4 = _make_edge_kernel(4, e_real, epw)
    ek1 = _make_edge_kernel(1, e_real, epw)

    h = x
    layers = [
        (4, ek4, W_l1, b_l1, W_r1, b_r1, att1, bias1, ln_g1, ln_b1),
        (4, ek4, W_l2, b_l2, W_r2, b_r2, att2, bias2, ln_g2, ln_b2),
        (1, ek1, W_l3, b_l3, W_r3, b_r3, att3, bias3, ln_g3, ln_b3),
    ]
    for heads, ek, Wl, bl, Wr, br, att, bias, g, b in layers:
        w = jnp.concatenate([Wl, Wr], axis=1)
        bb = jnp.concatenate([bl, br]).reshape(1, 2 * D)
        xl, xr = _mm_call(h, w, bb)
        acc = ek(edges, xl, xr, att.reshape(-1))
        h = _comb_call(acc, bias.reshape(1, D), g.reshape(1, D),
                       b.reshape(1, D), heads)
    return h


# speedup vs baseline: 59.5522x; 1.2275x over previous
import functools

import jax
import jax.numpy as jnp
from jax import lax
from jax.experimental import pallas as pl
from jax.experimental.pallas import tpu as pltpu
from jax.experimental.pallas import tpu_sc as plsc

N_NODES = 10000
N_PAD = 10240
D = 128
ACCW = 144
NC, NS, L = 2, 16, 16
NW = NC * NS
K = 40
ROWS_PER_SUB = N_PAD // NS
ZCH = 40
WCH = 128


def _make_edge_kernel(heads, e_real, epw):
    cpw = epw // K
    vph = (D // L) // heads
    mesh = plsc.VectorSubcoreMesh(core_axis_name="c", subcore_axis_name="s")

    @functools.partial(
        pl.kernel,
        mesh=mesh,
        out_type=jax.ShapeDtypeStruct((NC, N_PAD, ACCW), jnp.float32),
        compiler_params=pltpu.CompilerParams(
            use_tc_tiling_on_sc=False, needs_layout_passes=False),
        scratch_types=[
            pltpu.VMEM((2, K), jnp.int32),
            pltpu.VMEM((2, K), jnp.int32),
            pltpu.VMEM((K, D), jnp.float32),
            pltpu.VMEM((K, D), jnp.float32),
            pltpu.VMEM((K, D), jnp.float32),
            pltpu.VMEM((K, D), jnp.float32),
            pltpu.VMEM((K, ACCW), jnp.float32),
            pltpu.VMEM((K, ACCW), jnp.float32),
            pltpu.VMEM((K,), jnp.int32),
            pltpu.VMEM((K,), jnp.int32),
            pltpu.VMEM((D,), jnp.float32),
            pltpu.VMEM_SHARED((N_PAD, ACCW), jnp.float32),
            pltpu.SemaphoreType.DMA,
            pltpu.SemaphoreType.DMA,
            pltpu.SemaphoreType.DMA,
            pltpu.SemaphoreType.DMA,
            pltpu.SemaphoreType.DMA,
            pltpu.SemaphoreType.DMA,
        ],
    )
    def edge_kernel(edges_hbm, tabl_hbm, tabr_hbm, att_hbm,
                    out_hbm, idx0_v, idx1_v, xl0_v, xl1_v, xr0_v, xr1_v,
                    orow0_v, orow1_v, sidx0_v, sidx1_v, att_v, acc_sh,
                    sl0, sl1, sr0, sr1, ss0, ss1):
        cid = lax.axis_index("c")
        sid = lax.axis_index("s")
        wid = cid * NS + sid
        idx_v = (idx0_v, idx1_v)
        xl_v = (xl0_v, xl1_v)
        xr_v = (xr0_v, xr1_v)
        orow_v = (orow0_v, orow1_v)
        sidx_v = (sidx0_v, sidx1_v)
        sl = (sl0, sl1)
        sr = (sr0, sr1)
        ss = (ss0, ss1)

        zv = jnp.zeros((L,), jnp.float32)

        def zrow(i, carry):
            for j in range(ACCW // L):
                orow0_v[i, pl.ds(j * L, L)] = zv
            return carry

        lax.fori_loop(0, ZCH, zrow, 0)
        row0 = sid * ROWS_PER_SUB
        for t in range(ROWS_PER_SUB // ZCH):
            pltpu.sync_copy(orow0_v.at[pl.ds(0, ZCH)],
                            acc_sh.at[pl.ds(row0 + t * ZCH, ZCH)])
        plsc.subcore_barrier()

        pltpu.sync_copy(att_hbm, att_v)
        attv = [att_v[pl.ds(j * L, L)] for j in range(D // L)]
        lane = jnp.arange(L, dtype=jnp.int32)
        ebase = wid * epw

        def fetch(g, b):
            base = ebase + g * K
            pltpu.sync_copy(edges_hbm.at[:, pl.ds(base, K)], idx_v[b])
            pltpu.async_copy(tabl_hbm.at[idx_v[b].at[0]], xl_v[b], sl[b])
            pltpu.async_copy(tabr_hbm.at[idx_v[b].at[1]], xr_v[b], sr[b])

        def compute(g, b):
            pltpu.make_async_copy(
                tabl_hbm.at[idx_v[b].at[0]], xl_v[b], sl[b]).wait()
            pltpu.make_async_copy(
                tabr_hbm.at[idx_v[b].at[1]], xr_v[b], sr[b]).wait()

            @pl.when(g >= 2)
            def _wait_prev_scatter():
                pltpu.make_async_copy(
                    orow_v[b], acc_sh.at[sidx_v[b]], ss[b]).wait()

            base = ebase + g * K

            @plsc.parallel_loop(0, K, unroll=8)
            def edge_body(e):
                xl = [xl_v[b][e, pl.ds(j * L, L)] for j in range(D // L)]
                t = []
                for j in range(D // L):
                    s = xl[j] + xr_v[b][e, pl.ds(j * L, L)]
                    t.append(jnp.maximum(s, s * 0.2) * attv[j])
                valid = (base + e) < e_real
                pv = jnp.zeros((L,), jnp.float32)
                pbs = []
                for h in range(heads):
                    acc = t[h * vph]
                    for j in range(1, vph):
                        acc = acc + t[h * vph + j]
                    alpha = jnp.sum(acc)
                    alpha = jnp.where(valid, alpha, -1e30)
                    pb = jnp.exp(jnp.broadcast_to(alpha, (L,)))
                    pbs.append(pb)
                    pv = jnp.where(lane == h, pb, pv)
                for j in range(D // L):
                    orow_v[b][e, pl.ds(j * L, L)] = xl[j] * pbs[j // vph]
                orow_v[b][e, pl.ds(D, L)] = pv

            for off in (0, 16, 24):
                sidx_v[b][pl.ds(off, L)] = idx_v[b][1, pl.ds(off, L)]
            pltpu.async_copy(orow_v[b], acc_sh.at[sidx_v[b]], ss[b], add=True)

        fetch(0, 0)

        def pair_body(i, carry):
            g0 = 2 * i
            fetch(g0 + 1, 1)
            compute(g0, 0)
            fetch(jnp.minimum(g0 + 2, cpw - 1), 0)
            compute(g0 + 1, 1)
            return carry

        lax.fori_loop(0, cpw // 2, pair_body, 0)
        pltpu.make_async_copy(
            tabl_hbm.at[idx_v[0].at[0]], xl_v[0], sl[0]).wait()
        pltpu.make_async_copy(
            tabr_hbm.at[idx_v[0].at[1]], xr_v[0], sr[0]).wait()
        pltpu.make_async_copy(orow_v[0], acc_sh.at[sidx_v[0]], ss[0]).wait()
        pltpu.make_async_copy(orow_v[1], acc_sh.at[sidx_v[1]], ss[1]).wait()
        plsc.subcore_barrier()

        for t in range(ROWS_PER_SUB // WCH):
            r = row0 + t * WCH
            pltpu.sync_copy(acc_sh.at[pl.ds(r, WCH)],
                            out_hbm.at[cid, pl.ds(r, WCH)])

    return edge_kernel


_MM_BLK = 400


def _mm_body(x_ref, w_ref, b_ref, ol_ref, or_ref):
    y = jnp.dot(x_ref[...], w_ref[...], preferred_element_type=jnp.float32)
    y = y + b_ref[...]
    ol_ref[...] = y[:, :D]
    or_ref[...] = y[:, D:]


def _mm_call(x, w, b):
    grid = N_NODES // _MM_BLK
    return pl.pallas_call(
        _mm_body,
        grid=(grid,),
        in_specs=[
            pl.BlockSpec((_MM_BLK, D), lambda i: (i, 0)),
            pl.BlockSpec((D, 2 * D), lambda i: (0, 0)),
            pl.BlockSpec((1, 2 * D), lambda i: (0, 0)),
        ],
        out_specs=[
            pl.BlockSpec((_MM_BLK, D), lambda i: (i, 0)),
            pl.BlockSpec((_MM_BLK, D), lambda i: (i, 0)),
        ],
        out_shape=[
            jax.ShapeDtypeStruct((N_NODES, D), jnp.float32),
            jax.ShapeDtypeStruct((N_NODES, D), jnp.float32),
        ],
    )(x, w, b)


def _comb_body(acc_ref, bias_ref, g_ref, b_ref, o_ref, *, heads):
    s = acc_ref[0] + acc_ref[1]
    cph = D // heads
    parts = []
    for h in range(heads):
        den = s[:, D + h:D + h + 1] + 1e-16
        parts.append(s[:, h * cph:(h + 1) * cph] / den)
    y = jnp.concatenate(parts, axis=1) + bias_ref[...]
    mu = jnp.mean(y, axis=-1, keepdims=True)
    d = y - mu
    var = jnp.mean(d * d, axis=-1, keepdims=True)
    z = d / jnp.sqrt(var + 1e-5) * g_ref[...] + b_ref[...]
    o_ref[...] = jnp.maximum(z, 0.0)


def _comb_call(acc, bias, g, b, heads):
    grid = N_NODES // _MM_BLK
    return pl.pallas_call(
        functools.partial(_comb_body, heads=heads),
        grid=(grid,),
        in_specs=[
            pl.BlockSpec((2, _MM_BLK, ACCW), lambda i: (0, i, 0)),
            pl.BlockSpec((1, D), lambda i: (0, 0)),
            pl.BlockSpec((1, D), lambda i: (0, 0)),
            pl.BlockSpec((1, D), lambda i: (0, 0)),
        ],
        out_specs=pl.BlockSpec((_MM_BLK, D), lambda i: (i, 0)),
        out_shape=jax.ShapeDtypeStruct((N_NODES, D), jnp.float32),
    )(acc, bias, g, b)


def kernel(x, edge_index, W_l1, b_l1, W_r1, b_r1, att1, bias1, ln_g1, ln_b1,
           W_l2, b_l2, W_r2, b_r2, att2, bias2, ln_g2, ln_b2,
           W_l3, b_l3, W_r3, b_r3, att3, bias3, ln_g3, ln_b3):
    e_in = edge_index.shape[1]
    e_real = e_in + N_NODES
    epw = -(-e_real // (NW * 2 * K)) * 2 * K
    epad = NW * epw

    loops = jnp.arange(N_NODES, dtype=jnp.int32)
    src = jnp.concatenate([edge_index[0].astype(jnp.int32), loops])
    dst = jnp.concatenate([edge_index[1].astype(jnp.int32), loops])
    edges = jnp.stack([jnp.pad(src, (0, epad - e_real)),
                       jnp.pad(dst, (0, epad - e_real))])

    ek4 = _make_edge_kernel(4, e_real, epw)
    ek1 = _make_edge_kernel(1, e_real, epw)

    h = x
    layers = [
        (4, ek4, W_l1, b_l1, W_r1, b_r1, att1, bias1, ln_g1, ln_b1),
        (4, ek4, W_l2, b_l2, W_r2, b_r2, att2, bias2, ln_g2, ln_b2),
        (1, ek1, W_l3, b_l3, W_r3, b_r3, att3, bias3, ln_g3, ln_b3),
    ]
    for heads, ek, Wl, bl, Wr, br, att, bias, g, b in layers:
        w = jnp.concatenate([Wl, Wr], axis=1)
        bb = jnp.concatenate([bl, br]).reshape(1, 2 * D)
        xl, xr = _mm_call(h, w, bb)
        acc = ek(edges, xl, xr, att.reshape(-1))
        h = _comb_call(acc, bias.reshape(1, D), g.reshape(1, D),
                       b.reshape(1, D), heads)
    return h
